# baseline (speedup 1.0000x reference)
import functools

import jax
import jax.numpy as jnp
from jax import lax
from jax.experimental import pallas as pl
from jax.experimental.pallas import tpu as pltpu
from jax.experimental.pallas import tpu_sc as plsc

_NUM_CORES = 2
_NUM_SUBCORES = 16
_NUM_WORKERS = _NUM_CORES * _NUM_SUBCORES
_CHUNK_ROWS = 8
_CHUNK_COLS = 2048
_NBUF = 4


def kernel(inputs):
    B, R, C = inputs.shape
    workers_per_row = _NUM_WORKERS // B
    rpw = R // workers_per_row
    ncol = C // _CHUNK_COLS
    nchunks = (rpw // _CHUNK_ROWS) * ncol
    assert rpw % _CHUNK_ROWS == 0 and C % _CHUNK_COLS == 0

    perm = jax.random.permutation(jax.random.key(42), B)
    inv_perm = jnp.argsort(perm)
    wid = (
        jnp.arange(_NUM_SUBCORES, dtype=jnp.int32)[None, :] * _NUM_CORES
        + jnp.arange(_NUM_CORES, dtype=jnp.int32)[:, None]
    )
    dst_batch = inv_perm.astype(jnp.int32)[wid // workers_per_row]
    dst_batch = jnp.broadcast_to(
        dst_batch[:, :, None], (_NUM_CORES, _NUM_SUBCORES, 16)
    ).astype(jnp.int32)

    mesh = plsc.VectorSubcoreMesh(core_axis_name="c", subcore_axis_name="s")

    @functools.partial(
        pl.kernel,
        out_type=jax.ShapeDtypeStruct((B, R, C), jnp.float32),
        mesh=mesh,
        scratch_types=[
            pltpu.VMEM((16,), jnp.int32),
            pltpu.VMEM_SHARED(
                (_NUM_SUBCORES, _NBUF, _CHUNK_ROWS, _CHUNK_COLS), jnp.float32
            ),
            pltpu.SemaphoreType.DMA,
            *[pltpu.SemaphoreType.DMA for _ in range(2 * _NBUF)],
        ],
    )
    def run(in_hbm, dst_hbm, out_hbm, idx_v, shared, *sems):
        isem = sems[0]
        lsems = sems[1 : _NBUF + 1]
        ssems = sems[_NBUF + 1 :]
        cid = lax.axis_index("c")
        sid = lax.axis_index("s")
        bufs = [shared.at[sid, b] for b in range(_NBUF)]
        w = sid * _NUM_CORES + cid
        src_b = w // workers_per_row
        r0 = (w % workers_per_row) * rpw

        idx_cp = pltpu.async_copy(dst_hbm.at[cid, sid], idx_v, isem)

        def src_at(k):
            rc = k // ncol
            cc = k % ncol
            return in_hbm.at[
                src_b,
                pl.ds(pl.multiple_of(r0 + rc * _CHUNK_ROWS, 8), _CHUNK_ROWS),
                pl.ds(pl.multiple_of(cc * _CHUNK_COLS, 128), _CHUNK_COLS),
            ]

        for b in range(_NBUF):
            pltpu.async_copy(src_at(b), bufs[b], lsems[b])

        idx_cp.wait()
        dst_b = idx_v[...][0]

        def dst_at(k):
            rc = k // ncol
            cc = k % ncol
            return out_hbm.at[
                dst_b,
                pl.ds(pl.multiple_of(r0 + rc * _CHUNK_ROWS, 8), _CHUNK_ROWS),
                pl.ds(pl.multiple_of(cc * _CHUNK_COLS, 128), _CHUNK_COLS),
            ]

        ngroups = nchunks // _NBUF

        def body(g, carry):
            for b in range(_NBUF):
                k = g * _NBUF + b
                pltpu.make_async_copy(src_at(k), bufs[b], lsems[b]).wait()
                st = pltpu.async_copy(bufs[b], dst_at(k), ssems[b])
                st.wait()

                @pl.when(g < ngroups - 1)
                def _():
                    pltpu.async_copy(src_at(k + _NBUF), bufs[b], lsems[b])

            return carry

        lax.fori_loop(0, ngroups, body, 0)

    return run(inputs, dst_batch)


# speedup vs baseline: 1.0237x; 1.0237x over previous
import functools

import jax
import jax.numpy as jnp
from jax import lax
from jax.experimental import pallas as pl
from jax.experimental.pallas import tpu as pltpu
from jax.experimental.pallas import tpu_sc as plsc

_NUM_CORES = 2
_NUM_SUBCORES = 16
_NUM_WORKERS = _NUM_CORES * _NUM_SUBCORES
_CHUNK_ROWS = 8
_CHUNK_COLS = 4096
_NBUF = 2


def kernel(inputs):
    B, R, C = inputs.shape
    workers_per_row = _NUM_WORKERS // B
    rpw = R // workers_per_row
    ncol = C // _CHUNK_COLS
    nchunks = (rpw // _CHUNK_ROWS) * ncol
    assert rpw % _CHUNK_ROWS == 0 and C % _CHUNK_COLS == 0

    perm = jax.random.permutation(jax.random.key(42), B)
    inv_perm = jnp.argsort(perm)
    wid = (
        jnp.arange(_NUM_SUBCORES, dtype=jnp.int32)[None, :] * _NUM_CORES
        + jnp.arange(_NUM_CORES, dtype=jnp.int32)[:, None]
    )
    dst_batch = inv_perm.astype(jnp.int32)[wid // workers_per_row]
    dst_batch = jnp.broadcast_to(
        dst_batch[:, :, None], (_NUM_CORES, _NUM_SUBCORES, 16)
    ).astype(jnp.int32)

    mesh = plsc.VectorSubcoreMesh(core_axis_name="c", subcore_axis_name="s")

    @functools.partial(
        pl.kernel,
        out_type=jax.ShapeDtypeStruct((B, R, C), jnp.float32),
        mesh=mesh,
        scratch_types=[
            pltpu.VMEM((16,), jnp.int32),
            pltpu.VMEM_SHARED(
                (_NUM_SUBCORES, _NBUF, _CHUNK_ROWS, _CHUNK_COLS), jnp.float32
            ),
            pltpu.SemaphoreType.DMA,
            *[pltpu.SemaphoreType.DMA for _ in range(2 * _NBUF)],
        ],
    )
    def run(in_hbm, dst_hbm, out_hbm, idx_v, shared, *sems):
        isem = sems[0]
        lsems = sems[1 : _NBUF + 1]
        ssems = sems[_NBUF + 1 :]
        cid = lax.axis_index("c")
        sid = lax.axis_index("s")
        bufs = [shared.at[sid, b] for b in range(_NBUF)]
        w = sid * _NUM_CORES + cid
        src_b = w // workers_per_row
        r0 = (w % workers_per_row) * rpw

        idx_cp = pltpu.async_copy(dst_hbm.at[cid, sid], idx_v, isem)

        def src_at(k):
            rc = k // ncol
            cc = k % ncol
            return in_hbm.at[
                src_b,
                pl.ds(pl.multiple_of(r0 + rc * _CHUNK_ROWS, 8), _CHUNK_ROWS),
                pl.ds(pl.multiple_of(cc * _CHUNK_COLS, 128), _CHUNK_COLS),
            ]

        for b in range(_NBUF):
            pltpu.async_copy(src_at(b), bufs[b], lsems[b])

        idx_cp.wait()
        dst_b = idx_v[...][0]

        def dst_at(k):
            rc = k // ncol
            cc = k % ncol
            return out_hbm.at[
                dst_b,
                pl.ds(pl.multiple_of(r0 + rc * _CHUNK_ROWS, 8), _CHUNK_ROWS),
                pl.ds(pl.multiple_of(cc * _CHUNK_COLS, 128), _CHUNK_COLS),
            ]

        ngroups = nchunks // _NBUF

        def body(g, carry):
            for b in range(_NBUF):
                k = g * _NBUF + b
                pltpu.make_async_copy(src_at(k), bufs[b], lsems[b]).wait()
                st = pltpu.async_copy(bufs[b], dst_at(k), ssems[b])
                st.wait()

                @pl.when(g < ngroups - 1)
                def _():
                    pltpu.async_copy(src_at(k + _NBUF), bufs[b], lsems[b])

            return carry

        lax.fori_loop(0, ngroups, body, 0)

    return run(inputs, dst_batch)
